# Initial kernel scaffold; baseline (speedup 1.0000x reference)
import jax
import jax.numpy as jnp
from jax.experimental import pallas as pl


def kernel(x, weight):
    raise NotImplementedError("write your pallas kernel here")


# speedup vs baseline: 2.9182x; 2.9182x over previous
import jax
import jax.numpy as jnp
from jax.experimental import pallas as pl
from jax.experimental.pallas import tpu as pltpu
from jax.experimental.pallas import tpu_sc as plsc

_MAX_NORM = 100.0
_EPS = 1e-7

_PRESCALE_BLOCK = 4000
_GATHER_WINDOW = 128


def _prescale_body(w_ref, o_ref):
    w = w_ref[...]
    norm = jnp.sqrt(jnp.sum(w * w, axis=1, keepdims=True))
    scale = jnp.where(norm > _MAX_NORM, _MAX_NORM / (norm + _EPS), 1.0)
    o_ref[...] = w * scale


def _prescale_table(weight):
    v, d = weight.shape
    return pl.pallas_call(
        _prescale_body,
        out_shape=jax.ShapeDtypeStruct((v, d), weight.dtype),
        grid=(v // _PRESCALE_BLOCK,),
        in_specs=[pl.BlockSpec((_PRESCALE_BLOCK, d), lambda i: (i, 0))],
        out_specs=pl.BlockSpec((_PRESCALE_BLOCK, d), lambda i: (i, 0)),
    )(weight)


def _sc_gather(table, idx_flat):
    num_idx = idx_flat.shape[0]
    d = table.shape[1]
    idx2d = idx_flat.reshape(1, num_idx)
    mesh = plsc.VectorSubcoreMesh(core_axis_name="core",
                                  subcore_axis_name="subcore")

    @pl.kernel(
        out_type=jax.ShapeDtypeStruct((num_idx, d), table.dtype),
        mesh=mesh,
    )
    def gather_kernel(table_hbm, idx_hbm, out_hbm):
        def body(idx_vmem, out_vmem):
            pltpu.sync_copy(table_hbm.at[idx_vmem.at[0]], out_vmem)

        pltpu.emit_pipeline(
            body,
            grid=(num_idx // _GATHER_WINDOW,),
            in_specs=[pl.BlockSpec((1, _GATHER_WINDOW),
                                   index_map=lambda i: (0, i))],
            out_specs=[pl.BlockSpec((_GATHER_WINDOW, d),
                                    index_map=lambda i: (i, 0))],
            core_axis_name=("core", "subcore"),
            dimension_semantics=(pltpu.PARALLEL,),
        )(idx_hbm, out_hbm)

    return gather_kernel(table, idx2d)


def kernel(x, weight):
    b, s = x.shape
    scaled = _prescale_table(weight)
    out = _sc_gather(scaled, x.reshape(-1))
    return out.reshape(b, s, weight.shape[1])


# speedup vs baseline: 4.9716x; 1.7037x over previous
import jax
import jax.numpy as jnp
from jax.experimental import pallas as pl
from jax.experimental.pallas import tpu as pltpu
from jax.experimental.pallas import tpu_sc as plsc

_MAX_NORM = 100.0
_EPS = 1e-7

_PRESCALE_BLOCK = 4000
_GATHER_WINDOW = 128


def _prescale_body(w_ref, o_ref):
    w = w_ref[...]
    norm = jnp.sqrt(jnp.sum(w * w, axis=1, keepdims=True))
    scale = jnp.where(norm > _MAX_NORM, _MAX_NORM / (norm + _EPS), 1.0)
    o_ref[...] = w * scale


def _prescale_table(weight):
    v, d = weight.shape
    return pl.pallas_call(
        _prescale_body,
        out_shape=jax.ShapeDtypeStruct((v, d), weight.dtype),
        grid=(v // _PRESCALE_BLOCK,),
        in_specs=[pl.BlockSpec((_PRESCALE_BLOCK, d), lambda i: (i, 0))],
        out_specs=pl.BlockSpec((_PRESCALE_BLOCK, d), lambda i: (i, 0)),
    )(weight)


def _sc_gather(table, x):
    b, s = x.shape
    d = table.shape[1]
    idx3d = x.reshape(b, 1, s)
    mesh = plsc.VectorSubcoreMesh(core_axis_name="core",
                                  subcore_axis_name="subcore")

    rows_per_step = 8

    @pl.kernel(
        out_type=jax.ShapeDtypeStruct((b, s, d), table.dtype),
        mesh=mesh,
        scratch_types=[pltpu.SemaphoreType.DMA],
    )
    def gather_kernel(table_hbm, idx_hbm, out_hbm, sem):
        def body(idx_vmem, out_vmem):
            copies = [
                pltpu.async_copy(table_hbm.at[idx_vmem.at[r, 0]],
                                 out_vmem.at[r], sem)
                for r in range(rows_per_step)
            ]
            for c in copies:
                c.wait()

        pltpu.emit_pipeline(
            body,
            grid=(b // rows_per_step,),
            in_specs=[pl.BlockSpec((rows_per_step, 1, s),
                                   index_map=lambda i: (i, 0, 0))],
            out_specs=[pl.BlockSpec((rows_per_step, s, d),
                                    index_map=lambda i: (i, 0, 0))],
            core_axis_name=("core", "subcore"),
            dimension_semantics=(pltpu.PARALLEL,),
        )(idx_hbm, out_hbm)

    return gather_kernel(table, idx3d)


def kernel(x, weight):
    scaled = _prescale_table(weight)
    return _sc_gather(scaled, x)


# speedup vs baseline: 4.9801x; 1.0017x over previous
import jax
import jax.numpy as jnp
from jax.experimental import pallas as pl
from jax.experimental.pallas import tpu as pltpu
from jax.experimental.pallas import tpu_sc as plsc

_MAX_NORM = 100.0
_EPS = 1e-7

_PRESCALE_BLOCK = 4000
_GATHER_WINDOW = 128


def _prescale_body(w_ref, o_ref):
    w = w_ref[...]
    norm = jnp.sqrt(jnp.sum(w * w, axis=1, keepdims=True))
    scale = jnp.where(norm > _MAX_NORM, _MAX_NORM / (norm + _EPS), 1.0)
    o_ref[...] = w * scale


def _prescale_table(weight):
    v, d = weight.shape
    return pl.pallas_call(
        _prescale_body,
        out_shape=jax.ShapeDtypeStruct((v, d), weight.dtype),
        grid=(v // _PRESCALE_BLOCK,),
        in_specs=[pl.BlockSpec((_PRESCALE_BLOCK, d), lambda i: (i, 0))],
        out_specs=pl.BlockSpec((_PRESCALE_BLOCK, d), lambda i: (i, 0)),
    )(weight)


def _sc_gather(table, x):
    b, s = x.shape
    d = table.shape[1]
    mesh = plsc.VectorSubcoreMesh(core_axis_name="core",
                                  subcore_axis_name="subcore")

    rows_per_step = 8

    @pl.kernel(
        out_type=jax.ShapeDtypeStruct((b, s, d), table.dtype),
        mesh=mesh,
        scratch_types=[pltpu.SemaphoreType.DMA],
    )
    def gather_kernel(table_hbm, idx_hbm, out_hbm, sem):
        def body(idx_vmem, out_vmem):
            copies = [
                pltpu.async_copy(table_hbm.at[idx_vmem.at[r]],
                                 out_vmem.at[r], sem)
                for r in range(rows_per_step)
            ]
            for c in copies:
                c.wait()

        pltpu.emit_pipeline(
            body,
            grid=(b // rows_per_step,),
            in_specs=[pl.BlockSpec((rows_per_step, s),
                                   index_map=lambda i: (i, 0))],
            out_specs=[pl.BlockSpec((rows_per_step, s, d),
                                    index_map=lambda i: (i, 0, 0))],
            core_axis_name=("core", "subcore"),
            dimension_semantics=(pltpu.PARALLEL,),
        )(idx_hbm, out_hbm)

    return gather_kernel(table, x)


def kernel(x, weight):
    scaled = _prescale_table(weight)
    return _sc_gather(scaled, x)


# speedup vs baseline: 5.1302x; 1.0301x over previous
import jax
from jax import lax
import jax.numpy as jnp
from jax.experimental import pallas as pl
from jax.experimental.pallas import tpu as pltpu
from jax.experimental.pallas import tpu_sc as plsc

_MAX_NORM = 100.0
_EPS = 1e-7

_PRESCALE_BLOCK = 10000
_ROWS_PER_CHUNK = 8


def _prescale_body(w_ref, o_ref):
    w = w_ref[...]
    norm = jnp.sqrt(jnp.sum(w * w, axis=1, keepdims=True))
    scale = jnp.where(norm > _MAX_NORM, _MAX_NORM / (norm + _EPS), 1.0)
    o_ref[...] = w * scale


def _prescale_table(weight):
    v, d = weight.shape
    return pl.pallas_call(
        _prescale_body,
        out_shape=jax.ShapeDtypeStruct((v, d), weight.dtype),
        grid=(v // _PRESCALE_BLOCK,),
        in_specs=[pl.BlockSpec((_PRESCALE_BLOCK, d), lambda i: (i, 0))],
        out_specs=pl.BlockSpec((_PRESCALE_BLOCK, d), lambda i: (i, 0)),
    )(weight)


def _sc_gather(table, x):
    b, s = x.shape
    d = table.shape[1]
    mesh = plsc.VectorSubcoreMesh(core_axis_name="core",
                                  subcore_axis_name="subcore")
    num_cores = 2
    num_subcores = 16
    num_workers = num_cores * num_subcores
    rows_per_worker = b // num_workers
    rc = _ROWS_PER_CHUNK
    n_chunks = rows_per_worker // rc

    @pl.kernel(
        out_type=jax.ShapeDtypeStruct((b, s, d), table.dtype),
        mesh=mesh,
        scratch_types=[
            pltpu.VMEM((rows_per_worker, s), jnp.int32),
            pltpu.VMEM((rc, s, d), jnp.float32),
            pltpu.VMEM((rc, s, d), jnp.float32),
            pltpu.SemaphoreType.DMA,
            pltpu.SemaphoreType.DMA,
            pltpu.SemaphoreType.DMA,
        ],
    )
    def gather_kernel(table_hbm, idx_hbm, out_hbm, idx_v, buf0, buf1,
                      gsem, osem0, osem1):
        wid = lax.axis_index("subcore") * num_cores + lax.axis_index("core")
        base = wid * rows_per_worker
        pltpu.sync_copy(idx_hbm.at[pl.ds(base, rows_per_worker)], idx_v)

        bufs = (buf0, buf1)
        osems = (osem0, osem1)

        def fill(c, buf):
            copies = [
                pltpu.async_copy(table_hbm.at[idx_v.at[c * rc + r]],
                                 buf.at[r], gsem)
                for r in range(rc)
            ]
            for cp in copies:
                cp.wait()

        def drain_out(buf, osem):
            pltpu.make_async_copy(buf, out_hbm.at[pl.ds(base, rc)], osem).wait()

        def fire_out(c, buf, osem):
            pltpu.async_copy(buf, out_hbm.at[pl.ds(base + c * rc, rc)], osem)

        fill(0, buf0)
        fire_out(0, buf0, osem0)
        fill(1, buf1)
        fire_out(1, buf1, osem1)

        @pl.loop(2, n_chunks)
        def _(c):
            @pl.when(c % 2 == 0)
            def _():
                drain_out(buf0, osem0)
                fill(c, buf0)
                fire_out(c, buf0, osem0)

            @pl.when(c % 2 == 1)
            def _():
                drain_out(buf1, osem1)
                fill(c, buf1)
                fire_out(c, buf1, osem1)

        drain_out(buf0, osem0)
        drain_out(buf1, osem1)

    return gather_kernel(table, x)


def kernel(x, weight):
    scaled = _prescale_table(weight)
    return _sc_gather(scaled, x)
